# baseline (speedup 1.0000x reference)
import functools

import jax
import jax.numpy as jnp
from jax import lax
from jax.experimental import pallas as pl
from jax.experimental.pallas import tpu as pltpu
from jax.experimental.pallas import tpu_sc as plsc

C = 200
P = 1024
N = 16384
D = 128
L = 16
NC = 2
NS = 16
NW = NC * NS
P_W = P // NW
N_W = N // NW
NCHUNK = N_W // 128
DC = D // L
GPC = 128 // L
NGN = N_W // L
NGT = (N_W + P_W) // L
C_FULL = C // L
C_TAIL = C - C_FULL * L


def _body(ctx_idx_hbm, pos_idx_hbm, neg_idx_hbm, ctx_tab_hbm, cen_tab_hbm,
          pos_out_hbm, neg_out_hbm,
          ctx_idx_v, ctx_rows_v, part_v, allpart_v,
          pidx_v, nidx_v, row_v, res_v, sums_v, ctx_shared_v,
          sem_ctx, sem_gat, sem_out, sem_idx):
    cid = lax.axis_index("c")
    sid = lax.axis_index("s")
    wid = sid * NC + cid
    pbase = wid * P_W
    nbase = wid * N_W

    ctx_idx_v[0, :] = jnp.zeros((L,), jnp.int32)

    @pl.when(sid < C_FULL)
    def _stage_ctx_full():
        pltpu.sync_copy(ctx_idx_hbm.at[pl.ds(sid * L, L)], ctx_idx_v.at[0])

    @pl.when(sid == C_FULL)
    def _stage_ctx_tail():
        pltpu.sync_copy(ctx_idx_hbm.at[pl.ds(C_FULL * L, C_TAIL)],
                        ctx_idx_v.at[0, pl.ds(0, C_TAIL)])

    ctx_dma = pltpu.async_copy(ctx_tab_hbm.at[ctx_idx_v.at[0]],
                               ctx_rows_v.at[0], sem_ctx)

    nidx_dmas = [
        pltpu.async_copy(neg_idx_hbm.at[pl.ds(nbase + j * 128, 128)],
                         nidx_v.at[j], sem_idx)
        for j in range(NCHUNK)
    ]
    pidx_dma = pltpu.async_copy(pos_idx_hbm.at[pl.ds(pbase, P_W)],
                                pidx_v.at[0], sem_idx)
    for j in range(NCHUNK):
        nidx_dmas[j].wait()
        pltpu.async_copy(cen_tab_hbm.at[nidx_v.at[j]], row_v.at[j],
                         sem_gat.at[j])
    pidx_dma.wait()
    pltpu.async_copy(cen_tab_hbm.at[pidx_v.at[0]],
                     row_v.at[NCHUNK, pl.ds(0, P_W)], sem_gat.at[NCHUNK])

    @pl.when(sid <= C_FULL)
    def _ctx_partial():
        ctx_dma.wait()
        base_r = sid * L

        def crow(k, accs):
            wt = jnp.where(base_r + k < C, 1.0 / C, 0.0)
            return tuple(accs[c] + ctx_rows_v[0, k, pl.ds(c * L, L)] * wt
                         for c in range(DC))

        accs = lax.fori_loop(
            0, L, crow,
            tuple(jnp.zeros((L,), jnp.float32) for _ in range(DC)))
        for c in range(DC):
            part_v[0, pl.ds(c * L, L)] = accs[c]
        pltpu.sync_copy(part_v.at[0], ctx_shared_v.at[sid])

    @pl.when(sid > C_FULL)
    def _ctx_zero():
        for c in range(DC):
            part_v[0, pl.ds(c * L, L)] = jnp.zeros((L,), jnp.float32)
        pltpu.sync_copy(part_v.at[0], ctx_shared_v.at[sid])

    plsc.subcore_barrier()
    pltpu.sync_copy(ctx_shared_v, allpart_v)

    def psum(s, accs):
        return tuple(accs[c] + allpart_v[s, pl.ds(c * L, L)]
                     for c in range(DC))

    ctx_cs = lax.fori_loop(
        0, NS, psum, tuple(jnp.zeros((L,), jnp.float32) for _ in range(DC)))

    lane_iota = lax.iota(jnp.int32, L)

    def group(g, carry):
        @pl.when(jnp.logical_and(g % GPC == 0, g < NGN))
        def _wait_neg_chunk():
            pltpu.make_async_copy(cen_tab_hbm.at[nidx_v.at[0]], row_v.at[0],
                                  sem_gat.at[g // GPC]).wait()

        @pl.when(g == NGN)
        def _wait_pos():
            pltpu.make_async_copy(cen_tab_hbm.at[pidx_v.at[0]],
                                  row_v.at[NCHUNK, pl.ds(0, P_W)],
                                  sem_gat.at[NCHUNK]).wait()

        j = g // GPC
        rbase = (g % GPC) * L
        accs = [row_v[j, rbase + ll, pl.ds(0, L)] * ctx_cs[0]
                for ll in range(L)]
        for c in range(1, DC):
            for ll in range(L):
                accs[ll] = accs[ll] + (row_v[j, rbase + ll, pl.ds(c * L, L)]
                                       * ctx_cs[c])
        for ll in range(L):
            sums_v[ll, :] = accs[ll]
        cols = [plsc.load_gather(sums_v,
                                 [lane_iota, jnp.full((L,), c, jnp.int32)])
                for c in range(L)]
        while len(cols) > 1:
            cols = [cols[i] + cols[i + 1] for i in range(0, len(cols), 2)]
        tot = cols[0]
        e = jnp.exp(jnp.where(g < NGN, 1.0, -1.0) * tot)
        res_v[pl.ds(g * L, L)] = 1.0 / (1.0 + e)
        return carry

    lax.fori_loop(0, NGT, group, 0)

    out_pos = pltpu.async_copy(res_v.at[pl.ds(N_W, P_W)],
                               pos_out_hbm.at[0, pl.ds(pbase, P_W)], sem_out)
    pltpu.sync_copy(res_v.at[pl.ds(0, N_W)],
                    neg_out_hbm.at[0, pl.ds(nbase, N_W)])
    out_pos.wait()


@jax.jit
def _cbow_sc(context_idx, pos_idx, neg_idx, context_table, center_table):
    mesh = plsc.VectorSubcoreMesh(core_axis_name="c", subcore_axis_name="s")
    f = functools.partial(
        pl.kernel,
        out_type=(jax.ShapeDtypeStruct((1, P), jnp.float32),
                  jax.ShapeDtypeStruct((1, N), jnp.float32)),
        mesh=mesh,
        compiler_params=pltpu.CompilerParams(needs_layout_passes=False,
                                             disable_bounds_checks=True),
        scratch_types=[
            pltpu.VMEM((1, L), jnp.int32),
            pltpu.VMEM((1, L, D), jnp.float32),
            pltpu.VMEM((1, D), jnp.float32),
            pltpu.VMEM((NS, D), jnp.float32),
            pltpu.VMEM((1, P_W), jnp.int32),
            pltpu.VMEM((NCHUNK, 128), jnp.int32),
            pltpu.VMEM((NCHUNK + 1, 128, D), jnp.float32),
            pltpu.VMEM((N_W + P_W,), jnp.float32),
            pltpu.VMEM((L, L), jnp.float32),
            pltpu.VMEM_SHARED((NS, D), jnp.float32),
            pltpu.SemaphoreType.DMA,
            pltpu.SemaphoreType.DMA((NCHUNK + 1,)),
            pltpu.SemaphoreType.DMA,
            pltpu.SemaphoreType.DMA,
        ],
    )(_body)
    return f(context_idx, pos_idx, neg_idx, context_table, center_table)


def kernel(context_idx, pos_idx, neg_idx, context_table, center_table):
    return _cbow_sc(context_idx.astype(jnp.int32),
                    pos_idx.astype(jnp.int32),
                    neg_idx.astype(jnp.int32),
                    context_table, center_table)


# speedup vs baseline: 1.0092x; 1.0092x over previous
import functools

import jax
import jax.numpy as jnp
from jax import lax
from jax.experimental import pallas as pl
from jax.experimental.pallas import tpu as pltpu
from jax.experimental.pallas import tpu_sc as plsc

C = 200
P = 1024
N = 16384
D = 128
L = 16
NC = 2
NS = 16
NW = NC * NS
P_W = P // NW
N_W = N // NW
NCHUNK = N_W // 128
DC = D // L
GPC = 128 // L
C_FULL = C // L
C_TAIL = C - C_FULL * L


def _body(ctx_idx_hbm, pos_idx_hbm, neg_idx_hbm, ctx_tab_hbm, cen_tab_hbm,
          pos_out_hbm, neg_out_hbm,
          ctx_idx_v, ctx_rows_v, part_v, allpart_v,
          pidx_v, prow_v, pres_v,
          nidx_v, nrow_v, nres_v, sums_v, ctx_shared_v,
          sem_ctx, sem_pos, sem_neg, sem_idx):
    cid = lax.axis_index("c")
    sid = lax.axis_index("s")
    wid = sid * NC + cid
    pbase = wid * P_W
    nbase = wid * N_W

    ctx_idx_v[0, :] = jnp.zeros((L,), jnp.int32)

    @pl.when(sid < C_FULL)
    def _stage_ctx_full():
        pltpu.sync_copy(ctx_idx_hbm.at[pl.ds(sid * L, L)], ctx_idx_v.at[0])

    @pl.when(sid == C_FULL)
    def _stage_ctx_tail():
        pltpu.sync_copy(ctx_idx_hbm.at[pl.ds(C_FULL * L, C_TAIL)],
                        ctx_idx_v.at[0, pl.ds(0, C_TAIL)])

    ctx_dma = pltpu.async_copy(ctx_tab_hbm.at[ctx_idx_v.at[0]],
                               ctx_rows_v.at[0], sem_ctx)

    pidx_dma = pltpu.async_copy(pos_idx_hbm.at[pl.ds(pbase, P_W)],
                                pidx_v.at[0], sem_idx)
    nidx_dmas = [
        pltpu.async_copy(neg_idx_hbm.at[pl.ds(nbase + j * 128, 128)],
                         nidx_v.at[j], sem_idx)
        for j in range(NCHUNK)
    ]
    pidx_dma.wait()
    pos_dma = pltpu.async_copy(cen_tab_hbm.at[pidx_v.at[0]],
                               prow_v.at[0], sem_pos)
    neg_dmas = []
    for j in range(NCHUNK):
        nidx_dmas[j].wait()
        neg_dmas.append(
            pltpu.async_copy(cen_tab_hbm.at[nidx_v.at[j]], nrow_v.at[j],
                             sem_neg.at[j]))

    @pl.when(sid <= C_FULL)
    def _ctx_partial():
        ctx_dma.wait()
        base_r = sid * L

        def crow(k, accs):
            wt = jnp.where(base_r + k < C, 1.0 / C, 0.0)
            return tuple(accs[c] + ctx_rows_v[0, k, pl.ds(c * L, L)] * wt
                         for c in range(DC))

        accs = lax.fori_loop(
            0, L, crow,
            tuple(jnp.zeros((L,), jnp.float32) for _ in range(DC)))
        for c in range(DC):
            part_v[0, pl.ds(c * L, L)] = accs[c]
        pltpu.sync_copy(part_v.at[0], ctx_shared_v.at[sid])

    @pl.when(sid > C_FULL)
    def _ctx_zero():
        for c in range(DC):
            part_v[0, pl.ds(c * L, L)] = jnp.zeros((L,), jnp.float32)
        pltpu.sync_copy(part_v.at[0], ctx_shared_v.at[sid])

    plsc.subcore_barrier()
    pltpu.sync_copy(ctx_shared_v, allpart_v)

    def psum(s, accs):
        return tuple(accs[c] + allpart_v[s, pl.ds(c * L, L)]
                     for c in range(DC))

    ctx_cs = lax.fori_loop(
        0, NS, psum, tuple(jnp.zeros((L,), jnp.float32) for _ in range(DC)))

    lane_iota = lax.iota(jnp.int32, L)

    def score_group(rows_ref, j, q, res_ref, res_off, neg):
        accs = [rows_ref[j, q * L + ll, pl.ds(0, L)] * ctx_cs[0]
                for ll in range(L)]
        for c in range(1, DC):
            for ll in range(L):
                accs[ll] = accs[ll] + (rows_ref[j, q * L + ll, pl.ds(c * L, L)]
                                       * ctx_cs[c])
        for ll in range(L):
            sums_v[ll, :] = accs[ll]
        cols = [plsc.load_gather(sums_v,
                                 [lane_iota, jnp.full((L,), c, jnp.int32)])
                for c in range(L)]
        while len(cols) > 1:
            cols = [cols[i] + cols[i + 1] for i in range(0, len(cols), 2)]
        tot = cols[0]
        e = jnp.exp(tot) if neg else jnp.exp(-tot)
        res_ref[pl.ds(res_off, L)] = 1.0 / (1.0 + e)

    pos_dma.wait()

    def pgroup(q, carry):
        score_group(prow_v, 0, q, pres_v, q * L, neg=False)
        return carry

    lax.fori_loop(0, P_W // L, pgroup, 0)
    out_pos_dma = pltpu.async_copy(pres_v, pos_out_hbm.at[0, pl.ds(pbase, P_W)],
                                   sem_pos)

    def nchunk(j, carry):
        pltpu.make_async_copy(
            cen_tab_hbm.at[nidx_v.at[0]], nrow_v.at[0], sem_neg.at[j]).wait()

        def ngroup(q, carry2):
            score_group(nrow_v, j, q, nres_v, j * 128 + q * L, neg=True)
            return carry2

        lax.fori_loop(0, GPC, ngroup, 0)
        return carry

    lax.fori_loop(0, NCHUNK, nchunk, 0)
    out_pos_dma.wait()
    pltpu.sync_copy(nres_v, neg_out_hbm.at[0, pl.ds(nbase, N_W)])


@jax.jit
def _cbow_sc(context_idx, pos_idx, neg_idx, context_table, center_table):
    mesh = plsc.VectorSubcoreMesh(core_axis_name="c", subcore_axis_name="s")
    f = functools.partial(
        pl.kernel,
        out_type=(jax.ShapeDtypeStruct((1, P), jnp.float32),
                  jax.ShapeDtypeStruct((1, N), jnp.float32)),
        mesh=mesh,
        compiler_params=pltpu.CompilerParams(needs_layout_passes=False,
                                             disable_bounds_checks=True,
                                             disable_semaphore_checks=True),
        scratch_types=[
            pltpu.VMEM((1, L), jnp.int32),
            pltpu.VMEM((1, L, D), jnp.float32),
            pltpu.VMEM((1, D), jnp.float32),
            pltpu.VMEM((NS, D), jnp.float32),
            pltpu.VMEM((1, P_W), jnp.int32),
            pltpu.VMEM((1, P_W, D), jnp.float32),
            pltpu.VMEM((P_W,), jnp.float32),
            pltpu.VMEM((NCHUNK, 128), jnp.int32),
            pltpu.VMEM((NCHUNK, 128, D), jnp.float32),
            pltpu.VMEM((N_W,), jnp.float32),
            pltpu.VMEM((L, L), jnp.float32),
            pltpu.VMEM_SHARED((NS, D), jnp.float32),
            pltpu.SemaphoreType.DMA,
            pltpu.SemaphoreType.DMA,
            pltpu.SemaphoreType.DMA((NCHUNK,)),
            pltpu.SemaphoreType.DMA,
        ],
    )(_body)
    return f(context_idx, pos_idx, neg_idx, context_table, center_table)


def kernel(context_idx, pos_idx, neg_idx, context_table, center_table):
    return _cbow_sc(context_idx.astype(jnp.int32),
                    pos_idx.astype(jnp.int32),
                    neg_idx.astype(jnp.int32),
                    context_table, center_table)
